# baseline (speedup 1.0000x reference)
import jax
import jax.numpy as jnp
from jax import lax
from jax.experimental import pallas as pl
from jax.experimental.pallas import tpu as pltpu
from jax.experimental.pallas import tpu_sc as plsc

NC = 2
NS = 16
NW = NC * NS

B = 16384
DA = 128
DB = 64
N_ROWS = 100000

BPW = B // NW
CH = 128
NCH = BPW // CH


def _body(idxa_hbm, idxb_hbm, fa_hbm, fb2_hbm, out_a, ow_hbm,
          idxa_v, idxb_v, ab0, ab1, bb0, bb1,
          sga0, sga1, sgb0, sgb1, swa0, swa1, swb0, swb1):
    c = lax.axis_index("c")
    s = lax.axis_index("s")
    wid = s * NC + c
    base = wid * BPW
    irow = wid * NCH

    pltpu.sync_copy(idxa_hbm.at[pl.ds(irow, NCH)], idxa_v)
    pltpu.sync_copy(idxb_hbm.at[pl.ds(irow, NCH)], idxb_v)

    abufs, bbufs = [ab0, ab1], [bb0, bb1]
    sga, sgb = [sga0, sga1], [sgb0, sgb1]
    swa, swb = [swa0, swa1], [swb0, swb1]
    wa = [None, None]
    wb = [None, None]
    for j in range(NCH):
        k = j % 2
        if wa[k] is not None:
            wa[k].wait()
            wb[k].wait()
        ga = pltpu.async_copy(fa_hbm.at[idxa_v.at[j]], abufs[k], sga[k])
        gb = pltpu.async_copy(fb2_hbm.at[idxb_v.at[j]], bbufs[k], sgb[k])
        ga.wait()
        gb.wait()
        wa[k] = pltpu.async_copy(
            abufs[k], out_a.at[pl.ds(base + j * CH, CH)], swa[k])
        wb[k] = pltpu.async_copy(
            bbufs[k], ow_hbm.at[pl.ds(base + j * CH, CH)], swb[k])
    for d in wa + wb:
        if d is not None:
            d.wait()


@jax.jit
def kernel(indices, feats_a, feats_b):
    idx = indices[:, 0].astype(jnp.int32)
    fb2 = feats_b.reshape(N_ROWS // 2, 2 * DB)
    idxa = idx.reshape(NW * NCH, CH)
    idxb = (idx >> 1).reshape(NW * NCH, CH)
    mesh = plsc.VectorSubcoreMesh(core_axis_name="c", subcore_axis_name="s")
    out_a, ow = pl.kernel(
        _body,
        out_type=[
            jax.ShapeDtypeStruct((B, DA), jnp.float32),
            jax.ShapeDtypeStruct((B, 2 * DB), jnp.float32),
        ],
        mesh=mesh,
        scratch_types=[
            pltpu.VMEM((NCH, CH), jnp.int32),
            pltpu.VMEM((NCH, CH), jnp.int32),
            pltpu.VMEM((CH, DA), jnp.float32),
            pltpu.VMEM((CH, DA), jnp.float32),
            pltpu.VMEM((CH, 2 * DB), jnp.float32),
            pltpu.VMEM((CH, 2 * DB), jnp.float32),
        ] + [pltpu.SemaphoreType.DMA] * 8,
    )(idxa, idxb, feats_a, fb2)
    out_b = jnp.where((idx & 1)[:, None] == 1, ow[:, DB:], ow[:, :DB])
    return out_a, out_b


# speedup vs baseline: 1.1046x; 1.1046x over previous
import jax
import jax.numpy as jnp
from jax import lax
from jax.experimental import pallas as pl
from jax.experimental.pallas import tpu as pltpu
from jax.experimental.pallas import tpu_sc as plsc

NC = 2
NS = 16
NW = NC * NS

B = 16384
DA = 128
DB = 64
N_ROWS = 100000

BPW = B // NW
CH = 128
NCH = BPW // CH


def _make_body(width):
    def _gather_body(idx_hbm, tab_hbm, out_hbm, idx_v, rb0, rb1,
                     sg0, sg1, sw0, sw1):
        c = lax.axis_index("c")
        s = lax.axis_index("s")
        wid = s * NC + c
        base = wid * BPW

        pltpu.sync_copy(idx_hbm.at[pl.ds(wid * NCH, NCH)], idx_v)

        bufs = [rb0, rb1]
        sg = [sg0, sg1]
        sw = [sw0, sw1]
        wd = [None, None]
        for j in range(NCH):
            k = j % 2
            if wd[k] is not None:
                wd[k].wait()
            pltpu.async_copy(tab_hbm.at[idx_v.at[j]], bufs[k], sg[k]).wait()
            wd[k] = pltpu.async_copy(
                bufs[k], out_hbm.at[pl.ds(base + j * CH, CH)], sw[k])
        for d in wd:
            if d is not None:
                d.wait()
    return _gather_body


def _gather_call(idx2d, table, width, untiled):
    mesh = plsc.VectorSubcoreMesh(core_axis_name="c", subcore_axis_name="s")
    params = {}
    if untiled:
        params["compiler_params"] = pltpu.CompilerParams(
            use_tc_tiling_on_sc=False)
    return pl.kernel(
        _make_body(width),
        out_type=jax.ShapeDtypeStruct((B, width), jnp.float32),
        mesh=mesh,
        scratch_types=[
            pltpu.VMEM((NCH, CH), jnp.int32),
            pltpu.VMEM((CH, width), jnp.float32),
            pltpu.VMEM((CH, width), jnp.float32),
        ] + [pltpu.SemaphoreType.DMA] * 4,
        **params,
    )(idx2d, table)


@jax.jit
def kernel(indices, feats_a, feats_b):
    idx = indices[:, 0].astype(jnp.int32)
    idx2d = idx.reshape(NW * NCH, CH)
    out_a = _gather_call(idx2d, feats_a, DA, untiled=False)
    out_b = _gather_call(idx2d, feats_b, DB, untiled=True)
    return out_a, out_b


# speedup vs baseline: 1.1079x; 1.0030x over previous
import jax
import jax.numpy as jnp
from jax import lax
from jax.experimental import pallas as pl
from jax.experimental.pallas import tpu as pltpu
from jax.experimental.pallas import tpu_sc as plsc

NC = 2
NS = 16
NW = NC * NS

B = 16384
DA = 128
DB = 64
N_ROWS = 100000

BPW = B // NW
CH = 128
NCH = BPW // CH


def _make_body(width):
    def _gather_body(idx_hbm, tab_hbm, out_hbm, idx_v, rb0, rb1,
                     sg0, sg1, sw0, sw1):
        c = lax.axis_index("c")
        s = lax.axis_index("s")
        wid = s * NC + c
        base = wid * BPW

        pltpu.sync_copy(idx_hbm.at[pl.ds(wid * NCH, NCH)], idx_v)

        bufs = [rb0, rb1]
        sg = [sg0, sg1]
        sw = [sw0, sw1]
        wd = [None, None]
        for j in range(NCH):
            k = j % 2
            if wd[k] is not None:
                wd[k].wait()
            pltpu.async_copy(tab_hbm.at[idx_v.at[j]], bufs[k], sg[k]).wait()
            wd[k] = pltpu.async_copy(
                bufs[k], out_hbm.at[pl.ds(base + j * CH, CH)], sw[k])
        for d in wd:
            if d is not None:
                d.wait()
    return _gather_body


def _gather_call(idx2d, table, width, untiled):
    mesh = plsc.VectorSubcoreMesh(core_axis_name="c", subcore_axis_name="s")
    params = {"compiler_params": pltpu.CompilerParams(
        use_tc_tiling_on_sc=False if untiled else None,
        skip_device_barrier=True)}
    return pl.kernel(
        _make_body(width),
        out_type=jax.ShapeDtypeStruct((B, width), jnp.float32),
        mesh=mesh,
        scratch_types=[
            pltpu.VMEM((NCH, CH), jnp.int32),
            pltpu.VMEM((CH, width), jnp.float32),
            pltpu.VMEM((CH, width), jnp.float32),
        ] + [pltpu.SemaphoreType.DMA] * 4,
        **params,
    )(idx2d, table)


@jax.jit
def kernel(indices, feats_a, feats_b):
    idx = indices[:, 0].astype(jnp.int32)
    idx2d = idx.reshape(NW * NCH, CH)
    out_a = _gather_call(idx2d, feats_a, DA, untiled=False)
    out_b = _gather_call(idx2d, feats_b, DB, untiled=True)
    return out_a, out_b
